# Initial kernel scaffold; baseline (speedup 1.0000x reference)
import jax
import jax.numpy as jnp
from jax.experimental import pallas as pl


def kernel(vertices, reverse_map):
    raise NotImplementedError("write your pallas kernel here")


# speedup vs baseline: 1.8110x; 1.8110x over previous
import functools

import jax
import jax.numpy as jnp
from jax import lax
from jax.experimental import pallas as pl
from jax.experimental.pallas import tpu as pltpu
from jax.experimental.pallas import tpu_sc as plsc


def _pick_chunk(n_rows_per_batch: int, d: int) -> int:
    budget = 460_000
    best = 0
    for ch in range(16, n_rows_per_batch + 1, 16):
        if n_rows_per_batch % ch:
            continue
        if ch * d * 4 + ch * 4 <= budget:
            best = ch
    if best == 0:
        raise ValueError("no valid chunk size")
    return best


@functools.partial(jax.jit, static_argnames=("bsz", "n", "d", "ch"))
def _sc_gather(table, idx, bsz, n, d, ch):
    nchunks = (bsz * n) // ch
    chunks_per_batch = n // ch
    mesh = plsc.VectorSubcoreMesh(core_axis_name="c", subcore_axis_name="s")
    info = plsc.get_sparse_core_info()
    nw = info.num_cores * info.num_subcores

    @functools.partial(
        pl.kernel,
        out_type=jax.ShapeDtypeStruct((bsz * n, d), table.dtype),
        mesh=mesh,
        scratch_types=[
            pltpu.VMEM((ch,), jnp.int32),
            pltpu.VMEM((ch, d), table.dtype),
            pltpu.SemaphoreType.DMA,
        ],
    )
    def k(table_hbm, idx_hbm, out_hbm, idx_v, rows_v, sem):
        wid = lax.axis_index("s") * info.num_cores + lax.axis_index("c")
        iters = (nchunks + nw - 1) // nw

        @pl.loop(0, iters)
        def _(i):
            c = wid + i * nw

            @pl.when(c < nchunks)
            def _():
                base = c * ch
                boff = (c // chunks_per_batch) * n
                pltpu.sync_copy(idx_hbm.at[pl.ds(base, ch)], idx_v)

                @pl.loop(0, ch, step=16)
                def _(j):
                    sl = pl.ds(j, 16)
                    idx_v[sl] = idx_v[sl] + boff

                pltpu.async_copy(table_hbm.at[idx_v], rows_v, sem).wait()
                pltpu.sync_copy(rows_v, out_hbm.at[pl.ds(base, ch)])

    return k(table, idx)


def kernel(vertices, reverse_map):
    bsz, n, d = vertices.shape
    ch = _pick_chunk(n, d)
    table = vertices.reshape(bsz * n, d)
    idx = reverse_map.reshape(bsz * n).astype(jnp.int32)
    out = _sc_gather(table, idx, bsz, n, d, ch)
    return out.reshape(bsz, n, d)


# speedup vs baseline: 1.8508x; 1.0220x over previous
import functools

import jax
import jax.numpy as jnp
from jax import lax
from jax.experimental import pallas as pl
from jax.experimental.pallas import tpu as pltpu
from jax.experimental.pallas import tpu_sc as plsc


def _pick_chunk(n_rows_per_batch: int, d: int) -> int:
    budget = 230_000
    best = 0
    for ch in range(16, n_rows_per_batch + 1, 16):
        if n_rows_per_batch % ch:
            continue
        if ch * d * 4 + ch * 4 <= budget:
            best = ch
    if best == 0:
        raise ValueError("no valid chunk size")
    return best


@functools.partial(jax.jit, static_argnames=("bsz", "n", "d", "ch"))
def _sc_gather(table, idx, bsz, n, d, ch):
    nchunks = (bsz * n) // ch
    chunks_per_batch = n // ch
    mesh = plsc.VectorSubcoreMesh(core_axis_name="c", subcore_axis_name="s")
    info = plsc.get_sparse_core_info()
    nw = info.num_cores * info.num_subcores

    @functools.partial(
        pl.kernel,
        out_type=jax.ShapeDtypeStruct((bsz * n, d), table.dtype),
        mesh=mesh,
        scratch_types=[
            pltpu.VMEM((ch,), jnp.int32),
            pltpu.VMEM((ch,), jnp.int32),
            pltpu.VMEM((ch, d), table.dtype),
            pltpu.VMEM((ch, d), table.dtype),
            pltpu.SemaphoreType.DMA,
            pltpu.SemaphoreType.DMA,
            pltpu.SemaphoreType.DMA,
        ],
    )
    def k(table_hbm, idx_hbm, out_hbm, idx0, idx1, rows0, rows1,
          semg, sems0, sems1):
        wid = lax.axis_index("s") * info.num_cores + lax.axis_index("c")
        iters = (nchunks + nw - 1) // nw
        iters_r = iters + (iters % 2)
        bufs = ((idx0, rows0, sems0), (idx1, rows1, sems1))

        @pl.loop(0, iters_r, step=2)
        def _(i):
            for par, (idx_v, rows_v, sems) in enumerate(bufs):
                c = wid + (i + par) * nw

                @pl.when(c < nchunks)
                def _():
                    base = c * ch
                    boff = (c // chunks_per_batch) * n
                    pltpu.sync_copy(idx_hbm.at[pl.ds(base, ch)], idx_v)

                    @pl.loop(0, ch, step=16)
                    def _(j):
                        sl = pl.ds(j, 16)
                        idx_v[sl] = idx_v[sl] + boff

                    @pl.when(i + par >= 2)
                    def _():
                        pltpu.make_async_copy(
                            rows_v, out_hbm.at[pl.ds(0, ch)], sems).wait()

                    pltpu.async_copy(table_hbm.at[idx_v], rows_v, semg).wait()
                    pltpu.async_copy(rows_v, out_hbm.at[pl.ds(base, ch)], sems)

        for par, (idx_v, rows_v, sems) in enumerate(bufs):
            @pl.when(wid + par * nw < nchunks)
            def _():
                pltpu.make_async_copy(
                    rows_v, out_hbm.at[pl.ds(0, ch)], sems).wait()

    return k(table, idx)


def kernel(vertices, reverse_map):
    bsz, n, d = vertices.shape
    ch = _pick_chunk(n, d)
    table = vertices.reshape(bsz * n, d)
    idx = reverse_map.reshape(bsz * n).astype(jnp.int32)
    out = _sc_gather(table, idx, bsz, n, d, ch)
    return out.reshape(bsz, n, d)


# speedup vs baseline: 1.9895x; 1.0750x over previous
import functools

import jax
import jax.numpy as jnp
from jax import lax
from jax.experimental import pallas as pl
from jax.experimental.pallas import tpu as pltpu
from jax.experimental.pallas import tpu_sc as plsc


def _pick_chunk(n_rows_per_batch: int, d: int) -> int:
    budget = 230_000
    best = 0
    for ch in range(16, n_rows_per_batch + 1, 16):
        if n_rows_per_batch % ch:
            continue
        if ch * d * 4 + ch * 4 <= budget:
            best = ch
    if best == 0:
        raise ValueError("no valid chunk size")
    return best


@functools.partial(jax.jit, static_argnames=("bsz", "n", "d", "ch"))
def _sc_gather(table, idx, bsz, n, d, ch):
    nchunks = (bsz * n) // ch
    chunks_per_batch = n // ch
    mesh = plsc.VectorSubcoreMesh(core_axis_name="c", subcore_axis_name="s")
    info = plsc.get_sparse_core_info()
    nw = info.num_cores * info.num_subcores

    @functools.partial(
        pl.kernel,
        out_type=jax.ShapeDtypeStruct((bsz * n, d), table.dtype),
        mesh=mesh,
        scratch_types=[
            pltpu.VMEM((ch,), jnp.int32),
            pltpu.VMEM((ch,), jnp.int32),
            pltpu.VMEM((ch, d), table.dtype),
            pltpu.VMEM((ch, d), table.dtype),
            pltpu.SemaphoreType.DMA,
            pltpu.SemaphoreType.DMA,
            pltpu.SemaphoreType.DMA,
            pltpu.SemaphoreType.DMA,
            pltpu.SemaphoreType.DMA,
            pltpu.SemaphoreType.DMA,
        ],
    )
    def k(table_hbm, idx_hbm, out_hbm, idx0, idx1, rows0, rows1,
          semi0, semi1, semg0, semg1, sems0, sems1):
        wid = lax.axis_index("s") * info.num_cores + lax.axis_index("c")
        iters = (nchunks + nw - 1) // nw
        bufs = ((idx0, rows0, semi0, semg0, sems0),
                (idx1, rows1, semi1, semg1, sems1))

        def drain_scatter(rows_v, sems):
            pltpu.make_async_copy(rows_v, out_hbm.at[pl.ds(0, ch)], sems).wait()

        for par, (idx_v, rows_v, semi, semg, sems) in enumerate(bufs):
            c0 = wid + par * nw

            @pl.when(c0 < nchunks)
            def _():
                pltpu.async_copy(idx_hbm.at[pl.ds(c0 * ch, ch)], idx_v, semi)

        kmax = iters + 1
        kmax_r = kmax + (kmax % 2)

        @pl.loop(0, kmax_r, step=2)
        def _(i):
            for par, (idx_v, rows_v, semi, semg, sems) in enumerate(bufs):
                k_it = i + par
                c = wid + k_it * nw
                oidx_v, orows_v, osemi, osemg, osems = bufs[1 - par]

                @pl.when(c < nchunks)
                def _():
                    pltpu.make_async_copy(
                        idx_hbm.at[pl.ds(0, ch)], idx_v, semi).wait()
                    boff = (c // chunks_per_batch) * n

                    @pl.loop(0, ch, step=16)
                    def _(j):
                        sl = pl.ds(j, 16)
                        idx_v[sl] = idx_v[sl] + boff

                    @pl.when(k_it >= 2)
                    def _():
                        drain_scatter(rows_v, sems)

                    pltpu.async_copy(table_hbm.at[idx_v], rows_v, semg)

                @pl.when((k_it >= 1) & (c - nw < nchunks))
                def _():
                    pltpu.make_async_copy(
                        table_hbm.at[oidx_v], orows_v, osemg).wait()
                    pltpu.async_copy(
                        orows_v, out_hbm.at[pl.ds((c - nw) * ch, ch)], osems)

                    @pl.when(c + nw < nchunks)
                    def _():
                        pltpu.async_copy(
                            idx_hbm.at[pl.ds((c + nw) * ch, ch)],
                            oidx_v, osemi)

        for par, (idx_v, rows_v, semi, semg, sems) in enumerate(bufs):
            @pl.when(wid + par * nw < nchunks)
            def _():
                drain_scatter(rows_v, sems)

    return k(table, idx)


def kernel(vertices, reverse_map):
    bsz, n, d = vertices.shape
    ch = _pick_chunk(n, d)
    table = vertices.reshape(bsz * n, d)
    idx = reverse_map.reshape(bsz * n).astype(jnp.int32)
    out = _sc_gather(table, idx, bsz, n, d, ch)
    return out.reshape(bsz, n, d)


# speedup vs baseline: 1.9901x; 1.0003x over previous
import functools

import jax
import jax.numpy as jnp
from jax import lax
from jax.experimental import pallas as pl
from jax.experimental.pallas import tpu as pltpu
from jax.experimental.pallas import tpu_sc as plsc

_NBUF = 2


def _pick_chunk(n_rows_per_batch: int, d: int, nb: int) -> int:
    budget = 460_000 // nb
    best = 0
    for ch in range(16, n_rows_per_batch + 1, 16):
        if n_rows_per_batch % ch:
            continue
        if ch * d * 4 + ch * 4 <= budget:
            best = ch
    if best == 0:
        raise ValueError("no valid chunk size")
    return best


@functools.partial(jax.jit, static_argnames=("bsz", "n", "d", "ch", "nb"))
def _sc_gather(table, idx, bsz, n, d, ch, nb):
    nchunks = (bsz * n) // ch
    chunks_per_batch = n // ch
    mesh = plsc.VectorSubcoreMesh(core_axis_name="c", subcore_axis_name="s")
    info = plsc.get_sparse_core_info()
    nw = info.num_cores * info.num_subcores

    @functools.partial(
        pl.kernel,
        out_type=jax.ShapeDtypeStruct((bsz * n, d), table.dtype),
        mesh=mesh,
        scratch_types=(
            [pltpu.VMEM((ch,), jnp.int32) for _ in range(nb)]
            + [pltpu.VMEM((ch, d), table.dtype) for _ in range(nb)]
            + [pltpu.SemaphoreType.DMA for _ in range(3 * nb)]
        ),
    )
    def k(table_hbm, idx_hbm, out_hbm, *scr):
        idxs, rows = scr[0:nb], scr[nb:2 * nb]
        semi, semg, sems = scr[2 * nb:3 * nb], scr[3 * nb:4 * nb], scr[4 * nb:]
        bufs = tuple(zip(idxs, rows, semi, semg, sems))
        wid = lax.axis_index("s") * info.num_cores + lax.axis_index("c")
        iters = (nchunks + nw - 1) // nw

        def drain_scatter(rows_v, sem):
            pltpu.make_async_copy(rows_v, out_hbm.at[pl.ds(0, ch)], sem).wait()

        def idx_src(c):
            return idx_hbm.at[pl.ds(c * ch, ch)]

        for par, (idx_v, _, si, _, _) in enumerate(bufs):
            c0 = wid + par * nw

            @pl.when(c0 < nchunks)
            def _():
                pltpu.async_copy(idx_src(c0), idx_v, si)

        kmax = iters + 1
        kmax_r = kmax + (-kmax) % nb

        @pl.loop(0, kmax_r, step=nb)
        def _(i):
            for par in range(nb):
                k_it = i + par
                c = wid + k_it * nw
                idx_v, rows_v, si, sg, ss = bufs[par]
                pidx_v, prows_v, psi, psg, pss = bufs[(par - 1) % nb]

                @pl.when(c < nchunks)
                def _():
                    pltpu.make_async_copy(
                        idx_hbm.at[pl.ds(0, ch)], idx_v, si).wait()
                    boff = (c // chunks_per_batch) * n

                    @pl.loop(0, ch, step=16)
                    def _(j):
                        sl = pl.ds(j, 16)
                        idx_v[sl] = idx_v[sl] + boff

                    @pl.when(k_it >= nb)
                    def _():
                        drain_scatter(rows_v, ss)

                    pltpu.async_copy(table_hbm.at[idx_v], rows_v, sg)

                @pl.when((k_it >= 1) & (c - nw < nchunks))
                def _():
                    pltpu.make_async_copy(
                        table_hbm.at[pidx_v], prows_v, psg).wait()
                    pltpu.async_copy(
                        prows_v, out_hbm.at[pl.ds((c - nw) * ch, ch)], pss)

                    @pl.when(c + (nb - 1) * nw < nchunks)
                    def _():
                        pltpu.async_copy(
                            idx_src(c + (nb - 1) * nw), pidx_v, psi)

        for par, (_, rows_v, _, _, ss) in enumerate(bufs):
            @pl.when(wid + par * nw < nchunks)
            def _():
                drain_scatter(rows_v, ss)

    return k(table, idx)


def kernel(vertices, reverse_map):
    bsz, n, d = vertices.shape
    ch = _pick_chunk(n, d, _NBUF)
    table = vertices.reshape(bsz * n, d)
    idx = reverse_map.reshape(bsz * n).astype(jnp.int32)
    out = _sc_gather(table, idx, bsz, n, d, ch, _NBUF)
    return out.reshape(bsz, n, d)


# speedup vs baseline: 2.0107x; 1.0103x over previous
import functools

import jax
import jax.numpy as jnp
from jax import lax
from jax.experimental import pallas as pl
from jax.experimental.pallas import tpu as pltpu
from jax.experimental.pallas import tpu_sc as plsc

_NBUF = 3


def _pick_chunk(n_rows_per_batch: int, d: int, nb: int) -> int:
    budget = 460_000 // nb
    best = 0
    for ch in range(16, n_rows_per_batch + 1, 16):
        if n_rows_per_batch % ch:
            continue
        if ch * d * 4 + ch * 4 <= budget:
            best = ch
    if best == 0:
        raise ValueError("no valid chunk size")
    return best


@functools.partial(jax.jit, static_argnames=("bsz", "n", "d", "ch", "nb"))
def _sc_gather(table, idx, bsz, n, d, ch, nb):
    nchunks = (bsz * n) // ch
    chunks_per_batch = n // ch
    mesh = plsc.VectorSubcoreMesh(core_axis_name="c", subcore_axis_name="s")
    info = plsc.get_sparse_core_info()
    nw = info.num_cores * info.num_subcores

    @functools.partial(
        pl.kernel,
        out_type=jax.ShapeDtypeStruct((bsz * n, d), table.dtype),
        mesh=mesh,
        scratch_types=(
            [pltpu.VMEM((ch,), jnp.int32) for _ in range(nb)]
            + [pltpu.VMEM((ch, d), table.dtype) for _ in range(nb)]
            + [pltpu.SemaphoreType.DMA for _ in range(3 * nb)]
        ),
    )
    def k(table_hbm, idx_hbm, out_hbm, *scr):
        idxs, rows = scr[0:nb], scr[nb:2 * nb]
        semi, semg, sems = scr[2 * nb:3 * nb], scr[3 * nb:4 * nb], scr[4 * nb:]
        bufs = tuple(zip(idxs, rows, semi, semg, sems))
        wid = lax.axis_index("s") * info.num_cores + lax.axis_index("c")
        iters = (nchunks + nw - 1) // nw

        def drain_scatter(rows_v, sem):
            pltpu.make_async_copy(rows_v, out_hbm.at[pl.ds(0, ch)], sem).wait()

        def idx_src(c):
            return idx_hbm.at[pl.ds(c * ch, ch)]

        for par, (idx_v, _, si, _, _) in enumerate(bufs):
            c0 = wid + par * nw

            @pl.when(c0 < nchunks)
            def _():
                pltpu.async_copy(idx_src(c0), idx_v, si)

        kmax = iters + 1
        kmax_r = kmax + (-kmax) % nb

        @pl.loop(0, kmax_r, step=nb)
        def _(i):
            for par in range(nb):
                k_it = i + par
                c = wid + k_it * nw
                idx_v, rows_v, si, sg, ss = bufs[par]
                pidx_v, prows_v, psi, psg, pss = bufs[(par - 1) % nb]

                @pl.when(c < nchunks)
                def _():
                    pltpu.make_async_copy(
                        idx_hbm.at[pl.ds(0, ch)], idx_v, si).wait()
                    boff = (c // chunks_per_batch) * n

                    @pl.loop(0, ch, step=16)
                    def _(j):
                        sl = pl.ds(j, 16)
                        idx_v[sl] = idx_v[sl] + boff

                    @pl.when(k_it >= nb)
                    def _():
                        drain_scatter(rows_v, ss)

                    pltpu.async_copy(table_hbm.at[idx_v], rows_v, sg)

                @pl.when((k_it >= 1) & (c - nw < nchunks))
                def _():
                    pltpu.make_async_copy(
                        table_hbm.at[pidx_v], prows_v, psg).wait()
                    pltpu.async_copy(
                        prows_v, out_hbm.at[pl.ds((c - nw) * ch, ch)], pss)

                    @pl.when(c + (nb - 1) * nw < nchunks)
                    def _():
                        pltpu.async_copy(
                            idx_src(c + (nb - 1) * nw), pidx_v, psi)

        for par, (_, rows_v, _, _, ss) in enumerate(bufs):
            @pl.when(wid + par * nw < nchunks)
            def _():
                drain_scatter(rows_v, ss)

    return k(table, idx)


def kernel(vertices, reverse_map):
    bsz, n, d = vertices.shape
    ch = _pick_chunk(n, d, _NBUF)
    table = vertices.reshape(bsz * n, d)
    idx = reverse_map.reshape(bsz * n).astype(jnp.int32)
    out = _sc_gather(table, idx, bsz, n, d, ch, _NBUF)
    return out.reshape(bsz, n, d)
